# baseline (speedup 1.0000x reference)
import functools

import jax
import jax.numpy as jnp
from jax import lax
from jax.experimental import pallas as pl
from jax.experimental.pallas import tpu as pltpu
from jax.experimental.pallas import tpu_sc as plsc

N_NODES = 10000
N_EDGES = 320000
D_EDGE = 16
D_FEAT = 128
D_U = 16

NC = 2
NS = 16
NW = NC * NS
LANES = 16

GROUP = 128
NGROUPS = N_EDGES // GROUP
G_PER_CHUNK = 8
NHALF = 2
H_GROUPS = 1280
H_EDGES = H_GROUPS * GROUP
GROUPS_PER_TILE = H_GROUPS // NW
CHUNKS_PER_TILE = GROUPS_PER_TILE // G_PER_CHUNK
TABLE_ROWS = 10240
ROWS_PER_SUBCORE = TABLE_ROWS // NS
ACC_ROWS = ROWS_PER_SUBCORE // GROUP

_sc_mesh = plsc.VectorSubcoreMesh(core_axis_name="c", subcore_axis_name="s")


def _make_sc_aggregate(accumulate):
    scratch = [
        pltpu.VMEM((G_PER_CHUNK, GROUP), jnp.int32),
        pltpu.VMEM((G_PER_CHUNK, GROUP), jnp.int32),
        pltpu.VMEM((G_PER_CHUNK * GROUP, D_EDGE), jnp.float32),
        pltpu.VMEM((ROWS_PER_SUBCORE, D_EDGE), jnp.float32),
        pltpu.VMEM((ACC_ROWS, GROUP), jnp.int32),
        pltpu.VMEM_SHARED((TABLE_ROWS, D_EDGE), jnp.float32),
        pltpu.VMEM_SHARED((TABLE_ROWS, D_EDGE), jnp.float32),
        pltpu.SemaphoreType.DMA,
        pltpu.SemaphoreType.DMA,
    ]

    def body(idx_hbm, ea_hbm, *rest):
        if accumulate:
            (prev_hbm, out_hbm, ri_v, ci_v, ea_v, z_v, ai_v,
             trow_sh, tcol_sh, lsem, ssem) = rest
        else:
            (out_hbm, ri_v, ci_v, ea_v, z_v, ai_v,
             trow_sh, tcol_sh, lsem, ssem) = rest
        c = lax.axis_index("c")
        s = lax.axis_index("s")
        tile = c * NS + s

        @pl.loop(0, ROWS_PER_SUBCORE)
        def _(i):
            z_v[i, :] = jnp.zeros((LANES,), jnp.float32)

        zslc = pl.ds(s * ROWS_PER_SUBCORE, ROWS_PER_SUBCORE)
        pltpu.sync_copy(z_v, trow_sh.at[zslc])
        pltpu.sync_copy(z_v, tcol_sh.at[zslc])
        plsc.subcore_barrier()

        start = tile * GROUPS_PER_TILE

        @pl.loop(0, CHUNKS_PER_TILE)
        def _(ch):
            gbase = start + ch * G_PER_CHUNK
            loads = [
                pltpu.async_copy(idx_hbm.at[pl.ds(gbase, G_PER_CHUNK)],
                                 ri_v, lsem),
                pltpu.async_copy(idx_hbm.at[pl.ds(H_GROUPS + gbase,
                                                  G_PER_CHUNK)], ci_v, lsem),
                pltpu.async_copy(ea_hbm.at[pl.ds(gbase * GROUP,
                                                 G_PER_CHUNK * GROUP)],
                                 ea_v, lsem),
            ]
            for cp in loads:
                cp.wait()

            scats = []
            for j in range(G_PER_CHUNK):
                src = ea_v.at[pl.ds(j * GROUP, GROUP)]
                scats.append(pltpu.async_copy(
                    src, trow_sh.at[ri_v.at[j]], ssem, add=True))
                scats.append(pltpu.async_copy(
                    src, tcol_sh.at[ci_v.at[j]], ssem, add=True))
            for cp in scats:
                cp.wait()

        if accumulate:
            base = s * ROWS_PER_SUBCORE
            iota16 = lax.iota(jnp.int32, LANES)
            for r in range(ACC_ROWS):
                for k in range(GROUP // LANES):
                    ai_v[r, k * LANES:(k + 1) * LANES] = (
                        iota16 + (base + r * GROUP + k * LANES))
            for t, tbl in ((0, trow_sh), (1, tcol_sh)):
                pltpu.sync_copy(prev_hbm.at[c, t, zslc], z_v)
                accs = []
                for r in range(ACC_ROWS):
                    accs.append(pltpu.async_copy(
                        z_v.at[pl.ds(r * GROUP, GROUP)],
                        tbl.at[ai_v.at[r]], ssem, add=True))
                for cp in accs:
                    cp.wait()

        plsc.subcore_barrier()

        oslc = pl.ds(s * ROWS_PER_SUBCORE, ROWS_PER_SUBCORE)
        pltpu.sync_copy(trow_sh.at[oslc], out_hbm.at[c, 0, oslc])
        pltpu.sync_copy(tcol_sh.at[oslc], out_hbm.at[c, 1, oslc])

    return pl.kernel(
        body,
        out_type=jax.ShapeDtypeStruct((NC, 2, TABLE_ROWS, D_EDGE),
                                      jnp.float32),
        mesh=_sc_mesh,
        compiler_params=pltpu.CompilerParams(use_tc_tiling_on_sc=False),
        scratch_types=scratch,
    )


_sc_aggregate_first = _make_sc_aggregate(accumulate=False)
_sc_aggregate_next = _make_sc_aggregate(accumulate=True)


_BN = 2000


def _mlp_a_body(x_ref, u_ref, w0x_ref, w0u_ref, b0_ref, h_ref):
    h = jnp.dot(x_ref[...], w0x_ref[...], preferred_element_type=jnp.float32)
    h += jnp.dot(u_ref[...], w0u_ref[...], preferred_element_type=jnp.float32) + b0_ref[...]
    h_ref[...] = h


def _tc_mlp_a(x, u, w0x, w0u, b0):
    grid = (N_NODES // _BN,)
    return pl.pallas_call(
        _mlp_a_body,
        grid=grid,
        in_specs=[
            pl.BlockSpec((_BN, D_FEAT), lambda i: (i, 0)),
            pl.BlockSpec((1, D_U), lambda i: (0, 0)),
            pl.BlockSpec((D_FEAT, D_FEAT), lambda i: (0, 0)),
            pl.BlockSpec((D_U, D_FEAT), lambda i: (0, 0)),
            pl.BlockSpec((1, D_FEAT), lambda i: (0, 0)),
        ],
        out_specs=pl.BlockSpec((_BN, D_FEAT), lambda i: (i, 0)),
        out_shape=jax.ShapeDtypeStruct((N_NODES, D_FEAT), jnp.float32),
    )(x, u, w0x, w0u, b0)


def _mlp_b_body(parts_ref, h1_ref, w0cr_ref, w1_ref, b1_ref, o_ref):
    aggr = parts_ref[0, 0] + parts_ref[1, 0]
    aggc = parts_ref[0, 1] + parts_ref[1, 1]
    ag = jnp.concatenate([aggc, aggr], axis=1)
    h = h1_ref[...] + jnp.dot(ag, w0cr_ref[...],
                              preferred_element_type=jnp.float32)
    h = jnp.where(h >= 0, h, 0.2 * h)
    o_ref[...] = jnp.dot(h, w1_ref[...], preferred_element_type=jnp.float32) + b1_ref[...]


def _tc_mlp_b(parts, h1, w0cr, w1t, b1):
    grid = (N_NODES // _BN,)
    return pl.pallas_call(
        _mlp_b_body,
        grid=grid,
        in_specs=[
            pl.BlockSpec((NC, 2, _BN, D_EDGE), lambda i: (0, 0, i, 0)),
            pl.BlockSpec((_BN, D_FEAT), lambda i: (i, 0)),
            pl.BlockSpec((2 * D_EDGE, D_FEAT), lambda i: (0, 0)),
            pl.BlockSpec((D_FEAT, D_FEAT), lambda i: (0, 0)),
            pl.BlockSpec((1, D_FEAT), lambda i: (0, 0)),
        ],
        out_specs=pl.BlockSpec((_BN, D_FEAT), lambda i: (i, 0)),
        out_shape=jax.ShapeDtypeStruct((N_NODES, D_FEAT), jnp.float32),
    )(parts, h1, w0cr, w1t, b1)


@jax.jit
def kernel(x, edge_index, edge_attr, u, W0, b0, W1, b1):
    idx_g = edge_index.astype(jnp.int32).reshape(2 * NGROUPS, GROUP)

    hg = NGROUPS // NHALF
    he = N_EDGES // NHALF
    zpad_i = jnp.zeros((H_GROUPS - hg, GROUP), jnp.int32)
    zpad_e = jnp.zeros((H_EDGES - he, D_EDGE), jnp.float32)
    parts = None
    for h in range(NHALF):
        r0 = h * hg
        idx_h = jnp.concatenate(
            [idx_g[r0:r0 + hg], zpad_i,
             idx_g[NGROUPS + r0:NGROUPS + r0 + hg], zpad_i], axis=0)
        ea_h = jnp.concatenate(
            [edge_attr[h * he:(h + 1) * he], zpad_e], axis=0)
        if parts is None:
            parts = _sc_aggregate_first(idx_h, ea_h)
        else:
            parts = _sc_aggregate_next(idx_h, ea_h, parts)

    w0cr = W0[:, : 2 * D_EDGE].T
    w0x = W0[:, 2 * D_EDGE: 2 * D_EDGE + D_FEAT].T
    w0u = W0[:, 2 * D_EDGE + D_FEAT:].T
    h1 = _tc_mlp_a(x, u, w0x, w0u, b0.reshape(1, D_FEAT))
    return _tc_mlp_b(parts, h1, w0cr, W1.T, b1.reshape(1, D_FEAT))


# speedup vs baseline: 1.7149x; 1.7149x over previous
import functools

import jax
import jax.numpy as jnp
from jax import lax
from jax.experimental import pallas as pl
from jax.experimental.pallas import tpu as pltpu
from jax.experimental.pallas import tpu_sc as plsc

N_NODES = 10000
N_EDGES = 320000
D_EDGE = 16
D_FEAT = 128
D_U = 16

NC = 2
NS = 16
NW = NC * NS
LANES = 16

GROUP = 128
NGROUPS = N_EDGES // GROUP
G_PER_CHUNK = 8
FULL_CHUNKS = (NGROUPS // NW) // G_PER_CHUNK
BASE_GROUPS = NGROUPS // NW
REM_GROUPS = NGROUPS % NW
TABLE_ROWS = 10240
ROWS_PER_SUBCORE = TABLE_ROWS // NS

_sc_mesh = plsc.VectorSubcoreMesh(core_axis_name="c", subcore_axis_name="s")


@functools.partial(
    pl.kernel,
    out_type=jax.ShapeDtypeStruct((NC, 2, TABLE_ROWS, D_EDGE), jnp.float32),
    mesh=_sc_mesh,
    compiler_params=pltpu.CompilerParams(use_tc_tiling_on_sc=False),
    scratch_types=[
        pltpu.VMEM((G_PER_CHUNK, GROUP), jnp.int32),
        pltpu.VMEM((G_PER_CHUNK, GROUP), jnp.int32),
        pltpu.VMEM((G_PER_CHUNK * GROUP, D_EDGE), jnp.float32),
        pltpu.VMEM((ROWS_PER_SUBCORE, D_EDGE), jnp.float32),
        pltpu.VMEM_SHARED((TABLE_ROWS, D_EDGE), jnp.float32),
        pltpu.VMEM_SHARED((TABLE_ROWS, D_EDGE), jnp.float32),
        pltpu.SemaphoreType.DMA,
        pltpu.SemaphoreType.DMA,
    ],
)
def _sc_aggregate(idx_hbm, ea_hbm, out_hbm,
                  ri_v, ci_v, ea_v, z_v, trow_sh, tcol_sh, lsem, ssem):
    c = lax.axis_index("c")
    s = lax.axis_index("s")
    tile = c * NS + s

    @pl.loop(0, ROWS_PER_SUBCORE)
    def _(i):
        z_v[i, :] = jnp.zeros((LANES,), jnp.float32)

    zslc = pl.ds(s * ROWS_PER_SUBCORE, ROWS_PER_SUBCORE)
    pltpu.sync_copy(z_v, trow_sh.at[zslc])
    pltpu.sync_copy(z_v, tcol_sh.at[zslc])
    plsc.subcore_barrier()

    start = tile * BASE_GROUPS + jnp.minimum(tile, REM_GROUPS)
    tail = BASE_GROUPS - FULL_CHUNKS * G_PER_CHUNK + jnp.where(
        tile < REM_GROUPS, 1, 0)

    @pl.loop(0, FULL_CHUNKS)
    def _(ch):
        gbase = start + ch * G_PER_CHUNK
        loads = [
            pltpu.async_copy(idx_hbm.at[pl.ds(gbase, G_PER_CHUNK)], ri_v, lsem),
            pltpu.async_copy(idx_hbm.at[pl.ds(NGROUPS + gbase, G_PER_CHUNK)],
                             ci_v, lsem),
            pltpu.async_copy(ea_hbm.at[pl.ds(gbase * GROUP,
                                             G_PER_CHUNK * GROUP)], ea_v, lsem),
        ]
        for cp in loads:
            cp.wait()

        scats = []
        for j in range(G_PER_CHUNK):
            src = ea_v.at[pl.ds(j * GROUP, GROUP)]
            scats.append(
                pltpu.async_copy(src, trow_sh.at[ri_v.at[j]], ssem, add=True))
            scats.append(
                pltpu.async_copy(src, tcol_sh.at[ci_v.at[j]], ssem, add=True))
        for cp in scats:
            cp.wait()

    tbase = start + FULL_CHUNKS * G_PER_CHUNK

    @pl.loop(0, tail)
    def _(j):
        g = tbase + j
        pltpu.sync_copy(idx_hbm.at[pl.ds(g, 1)], ri_v.at[pl.ds(0, 1)])
        pltpu.sync_copy(idx_hbm.at[pl.ds(NGROUPS + g, 1)], ci_v.at[pl.ds(0, 1)])
        pltpu.sync_copy(ea_hbm.at[pl.ds(g * GROUP, GROUP)],
                        ea_v.at[pl.ds(0, GROUP)])
        src = ea_v.at[pl.ds(0, GROUP)]
        pltpu.sync_copy(src, trow_sh.at[ri_v.at[0]], add=True)
        pltpu.sync_copy(src, tcol_sh.at[ci_v.at[0]], add=True)

    plsc.subcore_barrier()

    oslc = pl.ds(s * ROWS_PER_SUBCORE, ROWS_PER_SUBCORE)
    pltpu.sync_copy(trow_sh.at[oslc], out_hbm.at[c, 0, oslc])
    pltpu.sync_copy(tcol_sh.at[oslc], out_hbm.at[c, 1, oslc])


_BN = 2000


def _mlp_a_body(x_ref, u_ref, w0x_ref, w0u_ref, b0_ref, h_ref):
    h = jnp.dot(x_ref[...], w0x_ref[...], preferred_element_type=jnp.float32)
    h += jnp.dot(u_ref[...], w0u_ref[...],
                 preferred_element_type=jnp.float32) + b0_ref[...]
    h_ref[...] = h


def _tc_mlp_a(x, u, w0x, w0u, b0):
    grid = (N_NODES // _BN,)
    return pl.pallas_call(
        _mlp_a_body,
        grid=grid,
        in_specs=[
            pl.BlockSpec((_BN, D_FEAT), lambda i: (i, 0)),
            pl.BlockSpec((1, D_U), lambda i: (0, 0)),
            pl.BlockSpec((D_FEAT, D_FEAT), lambda i: (0, 0)),
            pl.BlockSpec((D_U, D_FEAT), lambda i: (0, 0)),
            pl.BlockSpec((1, D_FEAT), lambda i: (0, 0)),
        ],
        out_specs=pl.BlockSpec((_BN, D_FEAT), lambda i: (i, 0)),
        out_shape=jax.ShapeDtypeStruct((N_NODES, D_FEAT), jnp.float32),
    )(x, u, w0x, w0u, b0)


def _mlp_b_body(parts_ref, h1_ref, w0cr_ref, w1_ref, b1_ref, o_ref):
    aggr = parts_ref[0, 0] + parts_ref[1, 0]
    aggc = parts_ref[0, 1] + parts_ref[1, 1]
    ag = jnp.concatenate([aggc, aggr], axis=1)
    h = h1_ref[...] + jnp.dot(ag, w0cr_ref[...],
                              preferred_element_type=jnp.float32)
    h = jnp.where(h >= 0, h, 0.2 * h)
    o_ref[...] = jnp.dot(h, w1_ref[...],
                         preferred_element_type=jnp.float32) + b1_ref[...]


def _tc_mlp_b(parts, h1, w0cr, w1t, b1):
    grid = (N_NODES // _BN,)
    return pl.pallas_call(
        _mlp_b_body,
        grid=grid,
        in_specs=[
            pl.BlockSpec((NC, 2, _BN, D_EDGE), lambda i: (0, 0, i, 0)),
            pl.BlockSpec((_BN, D_FEAT), lambda i: (i, 0)),
            pl.BlockSpec((2 * D_EDGE, D_FEAT), lambda i: (0, 0)),
            pl.BlockSpec((D_FEAT, D_FEAT), lambda i: (0, 0)),
            pl.BlockSpec((1, D_FEAT), lambda i: (0, 0)),
        ],
        out_specs=pl.BlockSpec((_BN, D_FEAT), lambda i: (i, 0)),
        out_shape=jax.ShapeDtypeStruct((N_NODES, D_FEAT), jnp.float32),
    )(parts, h1, w0cr, w1t, b1)


@jax.jit
def kernel(x, edge_index, edge_attr, u, W0, b0, W1, b1):
    idx_g = edge_index.astype(jnp.int32).reshape(2 * NGROUPS, GROUP)

    parts = _sc_aggregate(idx_g, edge_attr)

    w0cr = W0[:, : 2 * D_EDGE].T
    w0x = W0[:, 2 * D_EDGE: 2 * D_EDGE + D_FEAT].T
    w0u = W0[:, 2 * D_EDGE + D_FEAT:].T
    h1 = _tc_mlp_a(x, u, w0x, w0u, b0.reshape(1, D_FEAT))
    return _tc_mlp_b(parts, h1, w0cr, W1.T, b1.reshape(1, D_FEAT))


# speedup vs baseline: 1.8117x; 1.0565x over previous
import functools

import jax
import jax.numpy as jnp
from jax import lax
from jax.experimental import pallas as pl
from jax.experimental.pallas import tpu as pltpu
from jax.experimental.pallas import tpu_sc as plsc

N_NODES = 10000
N_EDGES = 320000
D_EDGE = 16
D_FEAT = 128
D_U = 16

NC = 2
NS = 16
NW = NC * NS
LANES = 16

GROUP = 128
NGROUPS = N_EDGES // GROUP
G_PER_CHUNK = 13
FULL_CHUNKS = (NGROUPS // NW) // G_PER_CHUNK
BASE_GROUPS = NGROUPS // NW
REM_GROUPS = NGROUPS % NW
TABLE_ROWS = 10240
ROWS_PER_SUBCORE = TABLE_ROWS // NS

_sc_mesh = plsc.VectorSubcoreMesh(core_axis_name="c", subcore_axis_name="s")


@functools.partial(
    pl.kernel,
    out_type=jax.ShapeDtypeStruct((NC, 2, TABLE_ROWS, D_EDGE), jnp.float32),
    mesh=_sc_mesh,
    compiler_params=pltpu.CompilerParams(use_tc_tiling_on_sc=False),
    scratch_types=[
        pltpu.VMEM((G_PER_CHUNK, GROUP), jnp.int32),
        pltpu.VMEM((G_PER_CHUNK, GROUP), jnp.int32),
        pltpu.VMEM((G_PER_CHUNK * GROUP, D_EDGE), jnp.float32),
        pltpu.VMEM((ROWS_PER_SUBCORE, D_EDGE), jnp.float32),
        pltpu.VMEM_SHARED((TABLE_ROWS, D_EDGE), jnp.float32),
        pltpu.VMEM_SHARED((TABLE_ROWS, D_EDGE), jnp.float32),
        pltpu.SemaphoreType.DMA,
        pltpu.SemaphoreType.DMA,
    ],
)
def _sc_aggregate(idx_hbm, ea_hbm, out_hbm,
                  ri_v, ci_v, ea_v, z_v, trow_sh, tcol_sh, lsem, ssem):
    c = lax.axis_index("c")
    s = lax.axis_index("s")
    tile = c * NS + s

    @pl.loop(0, ROWS_PER_SUBCORE)
    def _(i):
        z_v[i, :] = jnp.zeros((LANES,), jnp.float32)

    zslc = pl.ds(s * ROWS_PER_SUBCORE, ROWS_PER_SUBCORE)
    pltpu.sync_copy(z_v, trow_sh.at[zslc])
    pltpu.sync_copy(z_v, tcol_sh.at[zslc])
    plsc.subcore_barrier()

    start = tile * BASE_GROUPS + jnp.minimum(tile, REM_GROUPS)
    tail = BASE_GROUPS - FULL_CHUNKS * G_PER_CHUNK + jnp.where(
        tile < REM_GROUPS, 1, 0)

    @pl.loop(0, FULL_CHUNKS)
    def _(ch):
        gbase = start + ch * G_PER_CHUNK
        loads = [
            pltpu.async_copy(idx_hbm.at[pl.ds(gbase, G_PER_CHUNK)], ri_v, lsem),
            pltpu.async_copy(idx_hbm.at[pl.ds(NGROUPS + gbase, G_PER_CHUNK)],
                             ci_v, lsem),
            pltpu.async_copy(ea_hbm.at[pl.ds(gbase * GROUP,
                                             G_PER_CHUNK * GROUP)], ea_v, lsem),
        ]
        for cp in loads:
            cp.wait()

        scats = []
        for j in range(G_PER_CHUNK):
            src = ea_v.at[pl.ds(j * GROUP, GROUP)]
            scats.append(
                pltpu.async_copy(src, trow_sh.at[ri_v.at[j]], ssem, add=True))
            scats.append(
                pltpu.async_copy(src, tcol_sh.at[ci_v.at[j]], ssem, add=True))
        for cp in scats:
            cp.wait()

    tbase = start + FULL_CHUNKS * G_PER_CHUNK

    @pl.loop(0, tail)
    def _(j):
        g = tbase + j
        pltpu.sync_copy(idx_hbm.at[pl.ds(g, 1)], ri_v.at[pl.ds(0, 1)])
        pltpu.sync_copy(idx_hbm.at[pl.ds(NGROUPS + g, 1)], ci_v.at[pl.ds(0, 1)])
        pltpu.sync_copy(ea_hbm.at[pl.ds(g * GROUP, GROUP)],
                        ea_v.at[pl.ds(0, GROUP)])
        src = ea_v.at[pl.ds(0, GROUP)]
        pltpu.sync_copy(src, trow_sh.at[ri_v.at[0]], add=True)
        pltpu.sync_copy(src, tcol_sh.at[ci_v.at[0]], add=True)

    plsc.subcore_barrier()

    oslc = pl.ds(s * ROWS_PER_SUBCORE, ROWS_PER_SUBCORE)
    pltpu.sync_copy(trow_sh.at[oslc], out_hbm.at[c, 0, oslc])
    pltpu.sync_copy(tcol_sh.at[oslc], out_hbm.at[c, 1, oslc])


_BN = 2000


def _mlp_a_body(x_ref, u_ref, w0x_ref, w0u_ref, b0_ref, h_ref):
    h = jnp.dot(x_ref[...], w0x_ref[...], preferred_element_type=jnp.float32)
    h += jnp.dot(u_ref[...], w0u_ref[...],
                 preferred_element_type=jnp.float32) + b0_ref[...]
    h_ref[...] = h


def _tc_mlp_a(x, u, w0x, w0u, b0):
    grid = (N_NODES // _BN,)
    return pl.pallas_call(
        _mlp_a_body,
        grid=grid,
        in_specs=[
            pl.BlockSpec((_BN, D_FEAT), lambda i: (i, 0)),
            pl.BlockSpec((1, D_U), lambda i: (0, 0)),
            pl.BlockSpec((D_FEAT, D_FEAT), lambda i: (0, 0)),
            pl.BlockSpec((D_U, D_FEAT), lambda i: (0, 0)),
            pl.BlockSpec((1, D_FEAT), lambda i: (0, 0)),
        ],
        out_specs=pl.BlockSpec((_BN, D_FEAT), lambda i: (i, 0)),
        out_shape=jax.ShapeDtypeStruct((N_NODES, D_FEAT), jnp.float32),
    )(x, u, w0x, w0u, b0)


def _mlp_b_body(parts_ref, h1_ref, w0cr_ref, w1_ref, b1_ref, o_ref):
    aggr = parts_ref[0, 0] + parts_ref[1, 0]
    aggc = parts_ref[0, 1] + parts_ref[1, 1]
    ag = jnp.concatenate([aggc, aggr], axis=1)
    h = h1_ref[...] + jnp.dot(ag, w0cr_ref[...],
                              preferred_element_type=jnp.float32)
    h = jnp.where(h >= 0, h, 0.2 * h)
    o_ref[...] = jnp.dot(h, w1_ref[...],
                         preferred_element_type=jnp.float32) + b1_ref[...]


def _tc_mlp_b(parts, h1, w0cr, w1t, b1):
    grid = (N_NODES // _BN,)
    return pl.pallas_call(
        _mlp_b_body,
        grid=grid,
        in_specs=[
            pl.BlockSpec((NC, 2, _BN, D_EDGE), lambda i: (0, 0, i, 0)),
            pl.BlockSpec((_BN, D_FEAT), lambda i: (i, 0)),
            pl.BlockSpec((2 * D_EDGE, D_FEAT), lambda i: (0, 0)),
            pl.BlockSpec((D_FEAT, D_FEAT), lambda i: (0, 0)),
            pl.BlockSpec((1, D_FEAT), lambda i: (0, 0)),
        ],
        out_specs=pl.BlockSpec((_BN, D_FEAT), lambda i: (i, 0)),
        out_shape=jax.ShapeDtypeStruct((N_NODES, D_FEAT), jnp.float32),
    )(parts, h1, w0cr, w1t, b1)


@jax.jit
def kernel(x, edge_index, edge_attr, u, W0, b0, W1, b1):
    idx_g = edge_index.astype(jnp.int32).reshape(2 * NGROUPS, GROUP)

    parts = _sc_aggregate(idx_g, edge_attr)

    w0cr = W0[:, : 2 * D_EDGE].T
    w0x = W0[:, 2 * D_EDGE: 2 * D_EDGE + D_FEAT].T
    w0u = W0[:, 2 * D_EDGE + D_FEAT:].T
    h1 = _tc_mlp_a(x, u, w0x, w0u, b0.reshape(1, D_FEAT))
    return _tc_mlp_b(parts, h1, w0cr, W1.T, b1.reshape(1, D_FEAT))
